# baseline (speedup 1.0000x reference)
import functools

import jax
import jax.numpy as jnp
from jax import lax
from jax.experimental import pallas as pl
from jax.experimental.pallas import tpu as pltpu
from jax.experimental.pallas import tpu_sc as plsc

N = 10000
E = 320000
D = 128
NT = 2500
P_MAX = 10.0
TAU = 1.0
NEG = -1e30

NCORES = 2
NSUB = 16
NWORK = NCORES * NSUB
KE = 128
EK = 80
CPW = 128
EPW = EK * CPW
EPAD = NWORK * EPW
NPAD = 10240
RPT = 640

NTP = 2560
TRASH = 2559
CT = 160
PADK = 1024
TOT = PADK + RPT

_f32 = jnp.float32
_i32 = jnp.int32


_RB = 2000


def _dotT(x, w):
    return lax.dot_general(x.astype(jnp.bfloat16), w.astype(jnp.bfloat16),
                           (((1,), (1,)), ((), ())),
                           preferred_element_type=_f32)


def _dotvT(x, w):
    xb = x.astype(jnp.bfloat16).astype(_f32)
    wb = w.astype(jnp.bfloat16).astype(_f32)
    return jnp.sum(xb * wb, axis=1, keepdims=True)


def _leaky(h):
    return jnp.where(h >= 0, h, 0.01 * h)


def _mm3_body(x_ref, w1_ref, w2_ref, w3_ref, b1_ref, b3_ref,
              a_ref, b_ref, c_ref):
    x = x_ref[...]
    a_ref[...] = _dotT(x, w1_ref[...]) + b1_ref[...]
    b_ref[...] = _dotT(x, w2_ref[...])
    c_ref[...] = _dotT(x, w3_ref[...]) + b3_ref[...]


def _tc_mm3(y, w1, w2, w3, b1, b3):
    spec_x = pl.BlockSpec((_RB, D), lambda i: (i, 0))
    spec_w = pl.BlockSpec((D, D), lambda i: (0, 0))
    spec_b = pl.BlockSpec((1, D), lambda i: (0, 0))
    out = jax.ShapeDtypeStruct((N, D), _f32)
    return pl.pallas_call(
        _mm3_body,
        grid=(N // _RB,),
        in_specs=[spec_x, spec_w, spec_w, spec_w, spec_b, spec_b],
        out_specs=[spec_x, spec_x, spec_x],
        out_shape=[out, out, out],
    )(y, w1, w2, w3, b1.reshape(1, D), b3.reshape(1, D))


def _hmm3_body(s0_ref, s1_ref, dw0_ref, dw1_ref, bv_ref, c_ref,
               w1_ref, w2_ref, w3_ref, b1_ref, b3_ref,
               a_ref, b_ref, c2_ref):
    dw = dw0_ref[0] + dw1_ref[0]
    h = s0_ref[0] + s1_ref[0] - dw * bv_ref[...] + c_ref[...]
    h = _leaky(h)
    a_ref[...] = _dotT(h, w1_ref[...]) + b1_ref[...]
    b_ref[...] = _dotT(h, w2_ref[...])
    c2_ref[...] = _dotT(h, w3_ref[...]) + b3_ref[...]


def _tc_hmm3(aggs, dws, bv, c, w1, w2, w3, b1, b3):
    spec_x = pl.BlockSpec((_RB, D), lambda i: (i, 0))
    spec_s0 = pl.BlockSpec((1, _RB, D), lambda i: (0, i, 0))
    spec_s1 = pl.BlockSpec((1, _RB, D), lambda i: (1, i, 0))
    spec_d0 = pl.BlockSpec((1, _RB, 1), lambda i: (0, i, 0))
    spec_d1 = pl.BlockSpec((1, _RB, 1), lambda i: (1, i, 0))
    spec_w = pl.BlockSpec((D, D), lambda i: (0, 0))
    spec_b = pl.BlockSpec((1, D), lambda i: (0, 0))
    out = jax.ShapeDtypeStruct((N, D), _f32)
    return pl.pallas_call(
        _hmm3_body,
        grid=(N // _RB,),
        in_specs=[spec_s0, spec_s1, spec_d0, spec_d1, spec_x, spec_x,
                  spec_w, spec_w, spec_w, spec_b, spec_b],
        out_specs=[spec_x, spec_x, spec_x],
        out_shape=[out, out, out],
    )(aggs, aggs, dws, dws, bv, c,
      w1, w2, w3, b1.reshape(1, D), b3.reshape(1, D))


def _hlog_body(s0_ref, s1_ref, dw0_ref, dw1_ref, bv_ref, c_ref, wg_ref,
               h_ref, lg_ref):
    dw = dw0_ref[0] + dw1_ref[0]
    h = s0_ref[0] + s1_ref[0] - dw * bv_ref[...] + c_ref[...]
    h = _leaky(h)
    h_ref[...] = h
    lg_ref[...] = _dotvT(h, wg_ref[...]) * (1.0 / TAU)


def _tc_hlog(aggs, dws, bv, c, wg):
    spec_x = pl.BlockSpec((_RB, D), lambda i: (i, 0))
    spec_s0 = pl.BlockSpec((1, _RB, D), lambda i: (0, i, 0))
    spec_s1 = pl.BlockSpec((1, _RB, D), lambda i: (1, i, 0))
    spec_d0 = pl.BlockSpec((1, _RB, 1), lambda i: (0, i, 0))
    spec_d1 = pl.BlockSpec((1, _RB, 1), lambda i: (1, i, 0))
    spec_wg = pl.BlockSpec((1, D), lambda i: (0, 0))
    spec_lg = pl.BlockSpec((_RB, 1), lambda i: (i, 0))
    return pl.pallas_call(
        _hlog_body,
        grid=(N // _RB,),
        in_specs=[spec_s0, spec_s1, spec_d0, spec_d1, spec_x, spec_x, spec_wg],
        out_specs=[spec_x, spec_lg],
        out_shape=[jax.ShapeDtypeStruct((N, D), _f32),
                   jax.ShapeDtypeStruct((N, 1), _f32)],
    )(aggs, aggs, dws, dws, bv, c, wg)


def _p_body(ys_ref, cnt_ref, wp_ref, p_ref):
    tx = ys_ref[...] / jnp.maximum(cnt_ref[...], 1.0)
    z = _dotvT(tx, wp_ref[...])
    p_ref[...] = P_MAX * jax.nn.sigmoid(z)


def _tc_p(ysum, cnt, wp):
    return pl.pallas_call(
        _p_body,
        grid=(1,),
        in_specs=[pl.BlockSpec((NTP, D), lambda i: (0, 0)),
                  pl.BlockSpec((NTP, 1), lambda i: (0, 0)),
                  pl.BlockSpec((1, D), lambda i: (0, 0))],
        out_specs=pl.BlockSpec((NTP, 1), lambda i: (0, 0)),
        out_shape=jax.ShapeDtypeStruct((NTP, 1), _f32),
    )(ysum, cnt, wp)


def _edge_body(a_hbm, src_hbm, dst_hbm, ew_hbm,
               out_hbm, dw_hbm,
               agg_sp, dw_sp,
               sidx0, didx0, ewv0, rows0, pdix0, pew0,
               sidx1, didx1, ewv1, rows1, pdix1, pew1,
               sidx2, didx2, ewv2, rows2, pdix2, pew2,
               sidx3, didx3, ewv3, rows3, pdix3, pew3,
               gsem, ssem, isem):
    c = lax.axis_index("c")
    s = lax.axis_index("s")
    w = c * NSUB + s
    z16 = jnp.zeros((16,), _f32)
    B = ((sidx0, didx0, ewv0, rows0, pdix0, pew0),
         (sidx1, didx1, ewv1, rows1, pdix1, pew1),
         (sidx2, didx2, ewv2, rows2, pdix2, pew2),
         (sidx3, didx3, ewv3, rows3, pdix3, pew3))
    _CH = [(i * EK, EK) for i in range(RPT // EK)]
    if RPT % EK:
        _CH.append((RPT // EK * EK, RPT % EK))

    for r in range(EK):
        for d8 in range(D // 16):
            rows0[r, pl.ds(d8 * 16, 16)] = z16
    for d8 in range(EK // 16):
        ewv0[0, pl.ds(d8 * 16, 16)] = z16
    for off, n in _CH:
        r0 = pl.multiple_of(s * RPT + off, 8)
        pltpu.sync_copy(rows0.at[pl.ds(0, n), :], agg_sp.at[pl.ds(r0, n), :])
        pltpu.sync_copy(ewv0.at[0, pl.ds(0, n)], dw_sp.at[pl.ds(r0, n)])
    plsc.subcore_barrier()

    def issue_idx(j, b):
        base = pl.multiple_of(w * EPW + j * EK, 8)
        pltpu.async_copy(src_hbm.at[pl.ds(base, EK)], b[0].at[0], isem)
        pltpu.async_copy(dst_hbm.at[pl.ds(base, EK)], b[1].at[0], isem)
        pltpu.async_copy(ew_hbm.at[pl.ds(base, EK)], b[2].at[0], isem)

    def wait_idx(b):
        pltpu.make_async_copy(src_hbm.at[pl.ds(0, EK)], b[0].at[0], isem).wait()
        pltpu.make_async_copy(dst_hbm.at[pl.ds(0, EK)], b[1].at[0], isem).wait()
        pltpu.make_async_copy(ew_hbm.at[pl.ds(0, EK)], b[2].at[0], isem).wait()

    def issue_gather(b):
        pltpu.async_copy(a_hbm.at[b[0].at[0]], b[3], gsem)

    def wait_gather(b):
        pltpu.make_async_copy(a_hbm.at[b[0].at[0]], b[3], gsem).wait()

    def scale(b):
        ewv, rows = b[2], b[3]

        @pl.loop(0, EK // 16)
        def _sc(g):
            ew16 = ewv[0, pl.ds(g * 16, 16)]
            for lane in range(16):
                e = g * 16 + lane
                sc = ew16[lane]
                for d8 in range(D // 16):
                    sl = pl.ds(d8 * 16, 16)
                    rows[e, sl] = rows[e, sl] * sc

    def copy_priv(b):
        for g in range(EK // 16):
            sl = pl.ds(g * 16, 16)
            b[4][0, sl] = b[1][0, sl]
            b[5][0, sl] = b[2][0, sl]

    def issue_scatter(b):
        pltpu.async_copy(b[3], agg_sp.at[b[4].at[0]], ssem, add=True)
        pltpu.async_copy(b[5].at[0], dw_sp.at[b[4].at[0]], ssem, add=True)

    def wait_scatter(b):
        pltpu.make_async_copy(b[3], agg_sp.at[b[4].at[0]], ssem).wait()
        pltpu.make_async_copy(b[5].at[0], dw_sp.at[b[4].at[0]], ssem).wait()

    def step(j, cur, n2, w_scat, w_idx, i_gath, i_idx):
        if w_scat:
            wait_scatter(n2)
        if w_idx:
            wait_idx(n2)
        if i_gath:
            issue_gather(n2)
        wait_gather(cur)
        scale(cur)
        copy_priv(cur)
        issue_scatter(cur)
        if i_idx:
            issue_idx(j + 4, cur)

    issue_idx(0, B[0])
    issue_idx(1, B[1])
    issue_idx(2, B[2])
    issue_idx(3, B[3])
    wait_idx(B[0])
    issue_gather(B[0])
    wait_idx(B[1])
    issue_gather(B[1])

    @pl.loop(0, (CPW - 4) // 4)
    def _quad(jj):
        j0 = jj * 4

        @pl.when(jj > 0)
        def _():
            wait_scatter(B[2])
        step(j0, B[0], B[2], False, True, True, True)

        @pl.when(jj > 0)
        def _():
            wait_scatter(B[3])
        step(j0 + 1, B[1], B[3], False, True, True, True)

        step(j0 + 2, B[2], B[0], True, True, True, True)
        step(j0 + 3, B[3], B[1], True, True, True, True)

    step(CPW - 4, B[0], B[2], True, True, True, False)
    step(CPW - 3, B[1], B[3], True, True, True, False)
    step(CPW - 2, B[2], B[0], True, False, False, False)
    step(CPW - 1, B[3], B[1], True, False, False, False)
    wait_scatter(B[2])
    wait_scatter(B[3])

    plsc.subcore_barrier()
    for off, n in _CH:
        r0 = pl.multiple_of(s * RPT + off, 8)
        pltpu.sync_copy(agg_sp.at[pl.ds(r0, n), :], rows0.at[pl.ds(0, n), :])
        pltpu.sync_copy(rows0.at[pl.ds(0, n), :], out_hbm.at[c, pl.ds(r0, n), :])
    d0 = pl.multiple_of(s * RPT, 8)
    pltpu.sync_copy(dw_sp.at[pl.ds(d0, RPT)], dw_hbm.at[c, pl.ds(d0, RPT)])


_edge_call = functools.partial(
    pl.kernel,
    out_type=[jax.ShapeDtypeStruct((NCORES, NPAD, D), _f32),
              jax.ShapeDtypeStruct((NCORES, NPAD), _f32)],
    mesh=plsc.VectorSubcoreMesh(core_axis_name="c", subcore_axis_name="s",
                                num_cores=NCORES, num_subcores=NSUB),
    compiler_params=pltpu.CompilerParams(needs_layout_passes=False),
    scratch_types=[
        pltpu.VMEM_SHARED((NPAD, D), _f32),
        pltpu.VMEM_SHARED((NPAD,), _f32),
        pltpu.VMEM((1, EK), _i32),
        pltpu.VMEM((1, EK), _i32),
        pltpu.VMEM((1, EK), _f32),
        pltpu.VMEM((EK, D), _f32),
        pltpu.VMEM((1, EK), _i32),
        pltpu.VMEM((1, EK), _f32),
        pltpu.VMEM((1, EK), _i32),
        pltpu.VMEM((1, EK), _i32),
        pltpu.VMEM((1, EK), _f32),
        pltpu.VMEM((EK, D), _f32),
        pltpu.VMEM((1, EK), _i32),
        pltpu.VMEM((1, EK), _f32),
        pltpu.VMEM((1, EK), _i32),
        pltpu.VMEM((1, EK), _i32),
        pltpu.VMEM((1, EK), _f32),
        pltpu.VMEM((EK, D), _f32),
        pltpu.VMEM((1, EK), _i32),
        pltpu.VMEM((1, EK), _f32),
        pltpu.VMEM((1, EK), _i32),
        pltpu.VMEM((1, EK), _i32),
        pltpu.VMEM((1, EK), _f32),
        pltpu.VMEM((EK, D), _f32),
        pltpu.VMEM((1, EK), _i32),
        pltpu.VMEM((1, EK), _f32),
        pltpu.SemaphoreType.DMA,
        pltpu.SemaphoreType.DMA,
        pltpu.SemaphoreType.DMA,
    ],
)(_edge_body)


NBLK = NTP // KE


def _final_body(h_hbm, lg_hbm, t_hbm,
                gamma_hbm, ysum_hbm, cnt_hbm,
                m_all, m_fin_sp, ssum_sp, cnt_sp, ysum_sp,
                tpad, va, vb, m_loc, mm1, blk, t5, e5, one5, hbuf):
    w = lax.axis_index("s")
    z16 = jnp.zeros((16,), _f32)
    neg16 = jnp.full((16,), NEG, _f32)
    one16 = jnp.ones((16,), _f32)
    R5 = RPT // KE

    def _for_my_blocks(fn):
        fn(pl.multiple_of(w * KE, KE))

        @pl.when(w < NBLK - NSUB)
        def _():
            fn(pl.multiple_of((w + NSUB) * KE, KE))

    for g in range(KE // 16):
        blk[pl.ds(g * 16, 16)] = z16
    for r in range(KE):
        for d8 in range(D // 16):
            hbuf[r, pl.ds(d8 * 16, 16)] = z16

    def _zero(c0):
        pltpu.sync_copy(blk, ssum_sp.at[pl.ds(c0, KE)])
        pltpu.sync_copy(blk, cnt_sp.at[pl.ds(c0, KE)])
        pltpu.sync_copy(hbuf, ysum_sp.at[pl.ds(c0, KE), :])

    _for_my_blocks(_zero)
    plsc.subcore_barrier()

    for g in range(PADK // 16):
        sl = pl.ds(g * 16, 16)
        tpad[sl] = jnp.full((16,), -1, _i32)
        va[sl] = neg16
        vb[sl] = neg16
    tpad[pl.ds(TOT, 16)] = jnp.full((16,), -3, _i32)
    nb = pl.multiple_of(w * RPT, KE)
    pltpu.sync_copy(t_hbm.at[pl.ds(nb, RPT)], tpad.at[pl.ds(PADK, RPT)])
    pltpu.sync_copy(lg_hbm.at[pl.ds(nb, RPT)], va.at[pl.ds(PADK, RPT)])
    for r5 in range(R5):
        pltpu.sync_copy(t_hbm.at[pl.ds(nb + r5 * KE, KE)], t5.at[r5])

    for g in range(KE // 16):
        one5[0, pl.ds(g * 16, 16)] = one16
    for r5 in range(R5):
        pltpu.sync_copy(one5.at[0], cnt_sp.at[t5.at[r5]], add=True)
        pltpu.sync_copy(h_hbm.at[pl.ds(nb + r5 * KE, KE), :], hbuf)
        pltpu.sync_copy(hbuf, ysum_sp.at[t5.at[r5]], add=True)

    bufs = (va, vb)
    for rnd in range(10):
        k = 1 << rnd
        src = bufs[rnd % 2]
        dst = bufs[(rnd + 1) % 2]

        @pl.loop(0, RPT // 16)
        def _rmax(j, k=k, src=src, dst=dst):
            i0 = PADK + j * 16
            t_c = tpad[pl.ds(i0, 16)]
            t_p = tpad[pl.ds(i0 - k, 16)]
            v_c = src[pl.ds(i0, 16)]
            v_p = src[pl.ds(i0 - k, 16)]
            dst[pl.ds(i0, 16)] = jnp.maximum(
                v_c, jnp.where(t_p == t_c, v_p, NEG))

    for g in range(NTP // 16):
        m_loc[pl.ds(g * 16, 16)] = neg16

    @pl.loop(0, RPT // 16)
    def _scat(j):
        i0 = PADK + j * 16
        t_c = tpad[pl.ds(i0, 16)]
        t_n = tpad[pl.ds(i0 + 1, 16)]
        v = va[pl.ds(i0, 16)]
        plsc.store_scatter(m_loc, [t_c], v, mask=t_c != t_n)

    pltpu.sync_copy(m_loc, m_all.at[pl.ds(pl.multiple_of(w * NTP, KE), NTP)])
    plsc.subcore_barrier()

    def _merge(c0):
        for r in range(NSUB):
            pltpu.sync_copy(m_all.at[pl.ds(r * NTP + c0, KE)],
                            mm1.at[pl.ds(r * KE, KE)])
        for g in range(KE // 16):
            sl = pl.ds(g * 16, 16)
            acc = mm1[sl]
            for r in range(1, NSUB):
                acc = jnp.maximum(acc, mm1[pl.ds(r * KE + g * 16, 16)])
            blk[sl] = acc
        pltpu.sync_copy(blk, m_fin_sp.at[pl.ds(c0, KE)])

    _for_my_blocks(_merge)
    plsc.subcore_barrier()

    pltpu.sync_copy(m_fin_sp, m_loc)
    pltpu.sync_copy(lg_hbm.at[pl.ds(nb, RPT)], vb.at[pl.ds(PADK, RPT)])
    for r5 in range(R5):
        @pl.loop(0, KE // 16)
        def _e(j2, r5=r5):
            i0 = PADK + r5 * KE + j2 * 16
            t_c = tpad[pl.ds(i0, 16)]
            x = vb[pl.ds(i0, 16)]
            mt = plsc.load_gather(m_loc, [t_c])
            e5[r5, pl.ds(j2 * 16, 16)] = jnp.exp(x - mt)

    for r5 in range(R5):
        pltpu.sync_copy(e5.at[r5], ssum_sp.at[t5.at[r5]], add=True)
    plsc.subcore_barrier()

    pltpu.sync_copy(ssum_sp, m_loc)
    for r5 in range(R5):
        @pl.loop(0, KE // 16)
        def _g(j2, r5=r5):
            i0 = PADK + r5 * KE + j2 * 16
            t_c = tpad[pl.ds(i0, 16)]
            sg = plsc.load_gather(m_loc, [t_c])
            sl = pl.ds(j2 * 16, 16)
            e5[r5, sl] = e5[r5, sl] / (sg + 1e-16)
    for r5 in range(R5):
        pltpu.sync_copy(e5.at[r5], gamma_hbm.at[pl.ds(nb + r5 * KE, KE)])

    def _wb(c0):
        pltpu.sync_copy(ysum_sp.at[pl.ds(c0, KE), :], hbuf)
        pltpu.sync_copy(hbuf, ysum_hbm.at[pl.ds(c0, KE), :])
        pltpu.sync_copy(cnt_sp.at[pl.ds(c0, KE)], blk)
        pltpu.sync_copy(blk, cnt_hbm.at[pl.ds(c0, KE)])

    _for_my_blocks(_wb)


_final_call = functools.partial(
    pl.kernel,
    out_type=[jax.ShapeDtypeStruct((NPAD,), _f32),
              jax.ShapeDtypeStruct((NTP, D), _f32),
              jax.ShapeDtypeStruct((NTP,), _f32)],
    mesh=plsc.VectorSubcoreMesh(core_axis_name="c", subcore_axis_name="s",
                                num_cores=1, num_subcores=NSUB),
    compiler_params=pltpu.CompilerParams(needs_layout_passes=False),
    scratch_types=[
        pltpu.VMEM_SHARED((NSUB * NTP,), _f32),
        pltpu.VMEM_SHARED((NTP,), _f32),
        pltpu.VMEM_SHARED((NTP,), _f32),
        pltpu.VMEM_SHARED((NTP,), _f32),
        pltpu.VMEM_SHARED((NTP, D), _f32),
        pltpu.VMEM((TOT + 16,), _i32),
        pltpu.VMEM((TOT,), _f32),
        pltpu.VMEM((TOT,), _f32),
        pltpu.VMEM((NTP,), _f32),
        pltpu.VMEM((NSUB * KE,), _f32),
        pltpu.VMEM((KE,), _f32),
        pltpu.VMEM((RPT // KE, KE), _i32),
        pltpu.VMEM((RPT // KE, KE), _f32),
        pltpu.VMEM((1, KE), _f32),
        pltpu.VMEM((KE, D), _f32),
    ],
)(_final_body)


def kernel(y, edge_index, edge_weight, transmitters_index,
           l1_W1, l1_b1, l1_W2, l1_W3, l1_b3,
           l2_W1, l2_b1, l2_W2, l2_W3, l2_b3,
           Wg, Wp):
    src = edge_index[0]
    dst = edge_index[1]
    pad_e = EPAD - E
    srcp = jnp.concatenate([src, jnp.zeros((pad_e,), _i32)])
    dstp = jnp.concatenate([dst, jnp.zeros((pad_e,), _i32)])
    ewp = jnp.concatenate([edge_weight, jnp.zeros((pad_e,), _f32)])
    tpad = jnp.concatenate(
        [transmitters_index, jnp.full((NPAD - N,), TRASH, _i32)])

    a1, bv1, c1 = _tc_mm3(y, l1_W1, l1_W2, l1_W3, l1_b1, l1_b3)
    agg1, dw = _edge_call(a1, srcp, dstp, ewp)
    dwr = dw.reshape(NCORES, NPAD, 1)

    a2, bv2, c2 = _tc_hmm3(agg1, dwr, bv1, c1,
                           l2_W1, l2_W2, l2_W3, l2_b1, l2_b3)
    agg2, _ = _edge_call(a2, srcp, dstp, ewp)

    h2, logits = _tc_hlog(agg2, dwr, bv2, c2, Wg)
    h2p = jnp.pad(h2, ((0, NPAD - N), (0, 0)))
    lgp = jnp.pad(logits.reshape(-1), (0, NPAD - N))

    gamma_flat, ysum, cnt = _final_call(h2p, lgp, tpad)
    gamma = gamma_flat[:N].reshape(N, 1)
    p = _tc_p(ysum, cnt.reshape(NTP, 1), Wp)[:NT]
    return (p, gamma)


# speedup vs baseline: 2.1698x; 2.1698x over previous
import functools

import jax
import jax.numpy as jnp
from jax import lax
from jax.experimental import pallas as pl
from jax.experimental.pallas import tpu as pltpu
from jax.experimental.pallas import tpu_sc as plsc

N = 10000
E = 320000
D = 128
NT = 2500
P_MAX = 10.0
TAU = 1.0
NEG = -1e30

NCORES = 2
NSUB = 16
NWORK = NCORES * NSUB
KE = 128
EK = 112
CPW = 90
EPW = EK * CPW
EPAD = NWORK * EPW
NPAD = 10240
RPT = 640

NTP = 2560
TRASH = 2559
CT = 160
PADK = 1024
TOT = PADK + RPT

_f32 = jnp.float32
_i32 = jnp.int32


_RB = 2000


def _dotT(x, w):
    return lax.dot_general(x.astype(jnp.bfloat16), w.astype(jnp.bfloat16),
                           (((1,), (1,)), ((), ())),
                           preferred_element_type=_f32)


def _dotvT(x, w):
    xb = x.astype(jnp.bfloat16).astype(_f32)
    wb = w.astype(jnp.bfloat16).astype(_f32)
    return jnp.sum(xb * wb, axis=1, keepdims=True)


def _leaky(h):
    return jnp.where(h >= 0, h, 0.01 * h)


def _mm3_body(x_ref, w1_ref, w2_ref, w3_ref, b1_ref, b3_ref,
              a_ref, b_ref, c_ref):
    x = x_ref[...]
    a_ref[...] = _dotT(x, w1_ref[...]) + b1_ref[...]
    b_ref[...] = _dotT(x, w2_ref[...])
    c_ref[...] = _dotT(x, w3_ref[...]) + b3_ref[...]


def _tc_mm3(y, w1, w2, w3, b1, b3):
    spec_x = pl.BlockSpec((_RB, D), lambda i: (i, 0))
    spec_w = pl.BlockSpec((D, D), lambda i: (0, 0))
    spec_b = pl.BlockSpec((1, D), lambda i: (0, 0))
    out = jax.ShapeDtypeStruct((N, D), _f32)
    return pl.pallas_call(
        _mm3_body,
        grid=(N // _RB,),
        in_specs=[spec_x, spec_w, spec_w, spec_w, spec_b, spec_b],
        out_specs=[spec_x, spec_x, spec_x],
        out_shape=[out, out, out],
    )(y, w1, w2, w3, b1.reshape(1, D), b3.reshape(1, D))


def _hmm3_body(s0_ref, s1_ref, dw0_ref, dw1_ref, bv_ref, c_ref,
               w1_ref, w2_ref, w3_ref, b1_ref, b3_ref,
               a_ref, b_ref, c2_ref):
    dw = dw0_ref[0] + dw1_ref[0]
    h = s0_ref[0] + s1_ref[0] - dw * bv_ref[...] + c_ref[...]
    h = _leaky(h)
    a_ref[...] = _dotT(h, w1_ref[...]) + b1_ref[...]
    b_ref[...] = _dotT(h, w2_ref[...])
    c2_ref[...] = _dotT(h, w3_ref[...]) + b3_ref[...]


def _tc_hmm3(aggs, dws, bv, c, w1, w2, w3, b1, b3):
    spec_x = pl.BlockSpec((_RB, D), lambda i: (i, 0))
    spec_s0 = pl.BlockSpec((1, _RB, D), lambda i: (0, i, 0))
    spec_s1 = pl.BlockSpec((1, _RB, D), lambda i: (1, i, 0))
    spec_d0 = pl.BlockSpec((1, _RB, 1), lambda i: (0, i, 0))
    spec_d1 = pl.BlockSpec((1, _RB, 1), lambda i: (1, i, 0))
    spec_w = pl.BlockSpec((D, D), lambda i: (0, 0))
    spec_b = pl.BlockSpec((1, D), lambda i: (0, 0))
    out = jax.ShapeDtypeStruct((N, D), _f32)
    return pl.pallas_call(
        _hmm3_body,
        grid=(N // _RB,),
        in_specs=[spec_s0, spec_s1, spec_d0, spec_d1, spec_x, spec_x,
                  spec_w, spec_w, spec_w, spec_b, spec_b],
        out_specs=[spec_x, spec_x, spec_x],
        out_shape=[out, out, out],
    )(aggs, aggs, dws, dws, bv, c,
      w1, w2, w3, b1.reshape(1, D), b3.reshape(1, D))


def _hlog_body(s0_ref, s1_ref, dw0_ref, dw1_ref, bv_ref, c_ref, wg_ref,
               h_ref, lg_ref):
    dw = dw0_ref[0] + dw1_ref[0]
    h = s0_ref[0] + s1_ref[0] - dw * bv_ref[...] + c_ref[...]
    h = _leaky(h)
    h_ref[...] = h
    lg_ref[...] = _dotvT(h, wg_ref[...]) * (1.0 / TAU)


def _tc_hlog(aggs, dws, bv, c, wg):
    spec_x = pl.BlockSpec((_RB, D), lambda i: (i, 0))
    spec_s0 = pl.BlockSpec((1, _RB, D), lambda i: (0, i, 0))
    spec_s1 = pl.BlockSpec((1, _RB, D), lambda i: (1, i, 0))
    spec_d0 = pl.BlockSpec((1, _RB, 1), lambda i: (0, i, 0))
    spec_d1 = pl.BlockSpec((1, _RB, 1), lambda i: (1, i, 0))
    spec_wg = pl.BlockSpec((1, D), lambda i: (0, 0))
    spec_lg = pl.BlockSpec((_RB, 1), lambda i: (i, 0))
    return pl.pallas_call(
        _hlog_body,
        grid=(N // _RB,),
        in_specs=[spec_s0, spec_s1, spec_d0, spec_d1, spec_x, spec_x, spec_wg],
        out_specs=[spec_x, spec_lg],
        out_shape=[jax.ShapeDtypeStruct((N, D), _f32),
                   jax.ShapeDtypeStruct((N, 1), _f32)],
    )(aggs, aggs, dws, dws, bv, c, wg)


def _p_body(ys_ref, cnt_ref, wp_ref, p_ref):
    tx = ys_ref[...] / jnp.maximum(cnt_ref[...], 1.0)
    z = _dotvT(tx, wp_ref[...])
    p_ref[...] = P_MAX * jax.nn.sigmoid(z)


def _tc_p(ysum, cnt, wp):
    return pl.pallas_call(
        _p_body,
        grid=(1,),
        in_specs=[pl.BlockSpec((NTP, D), lambda i: (0, 0)),
                  pl.BlockSpec((NTP, 1), lambda i: (0, 0)),
                  pl.BlockSpec((1, D), lambda i: (0, 0))],
        out_specs=pl.BlockSpec((NTP, 1), lambda i: (0, 0)),
        out_shape=jax.ShapeDtypeStruct((NTP, 1), _f32),
    )(ysum, cnt, wp)


def _edge_body(a_hbm, src_hbm, dst_hbm, ew_hbm,
               out_hbm, dw_hbm,
               agg_sp, dw_sp,
               sidx0, didx0, ewv0, rows0, pdix0, pew0,
               sidx1, didx1, ewv1, rows1, pdix1, pew1,
               sidx2, didx2, ewv2, rows2, pdix2, pew2,
               gsem, ssem, isem):
    c = lax.axis_index("c")
    s = lax.axis_index("s")
    w = c * NSUB + s
    z16 = jnp.zeros((16,), _f32)
    B = ((sidx0, didx0, ewv0, rows0, pdix0, pew0),
         (sidx1, didx1, ewv1, rows1, pdix1, pew1),
         (sidx2, didx2, ewv2, rows2, pdix2, pew2))
    _CH = [(i * EK, EK) for i in range(RPT // EK)]
    if RPT % EK:
        _CH.append((RPT // EK * EK, RPT % EK))

    for r in range(EK):
        for d8 in range(D // 16):
            rows0[r, pl.ds(d8 * 16, 16)] = z16
    for d8 in range(EK // 16):
        ewv0[0, pl.ds(d8 * 16, 16)] = z16
    for off, n in _CH:
        r0 = pl.multiple_of(s * RPT + off, 8)
        pltpu.sync_copy(rows0.at[pl.ds(0, n), :], agg_sp.at[pl.ds(r0, n), :])
        pltpu.sync_copy(ewv0.at[0, pl.ds(0, n)], dw_sp.at[pl.ds(r0, n)])
    plsc.subcore_barrier()

    def issue_idx(j, b):
        base = pl.multiple_of(w * EPW + j * EK, 8)
        pltpu.async_copy(src_hbm.at[pl.ds(base, EK)], b[0].at[0], isem)
        pltpu.async_copy(dst_hbm.at[pl.ds(base, EK)], b[1].at[0], isem)
        pltpu.async_copy(ew_hbm.at[pl.ds(base, EK)], b[2].at[0], isem)

    def wait_idx(b):
        pltpu.make_async_copy(src_hbm.at[pl.ds(0, EK)], b[0].at[0], isem).wait()
        pltpu.make_async_copy(dst_hbm.at[pl.ds(0, EK)], b[1].at[0], isem).wait()
        pltpu.make_async_copy(ew_hbm.at[pl.ds(0, EK)], b[2].at[0], isem).wait()

    def issue_gather(b):
        pltpu.async_copy(a_hbm.at[b[0].at[0]], b[3], gsem)

    def wait_gather(b):
        pltpu.make_async_copy(a_hbm.at[b[0].at[0]], b[3], gsem).wait()

    def scale(b):
        ewv, rows = b[2], b[3]

        @pl.loop(0, EK // 16)
        def _sc(g):
            ew16 = ewv[0, pl.ds(g * 16, 16)]
            for lane in range(16):
                e = g * 16 + lane
                sc = ew16[lane]
                for d8 in range(D // 16):
                    sl = pl.ds(d8 * 16, 16)
                    rows[e, sl] = rows[e, sl] * sc

    def copy_priv(b):
        for g in range(EK // 16):
            sl = pl.ds(g * 16, 16)
            b[4][0, sl] = b[1][0, sl]
            b[5][0, sl] = b[2][0, sl]

    def issue_scatter(b):
        pltpu.async_copy(b[3], agg_sp.at[b[4].at[0]], ssem, add=True)
        pltpu.async_copy(b[5].at[0], dw_sp.at[b[4].at[0]], ssem, add=True)

    def wait_scatter(b):
        pltpu.make_async_copy(b[3], agg_sp.at[b[4].at[0]], ssem).wait()
        pltpu.make_async_copy(b[5].at[0], dw_sp.at[b[4].at[0]], ssem).wait()

    def step(j, cur, nxt, w_scat, w_idx, i_gath, i_idx):
        if w_scat:
            wait_scatter(nxt)
        if w_idx:
            wait_idx(nxt)
        if i_gath:
            issue_gather(nxt)
        wait_gather(cur)
        scale(cur)
        copy_priv(cur)
        issue_scatter(cur)
        if i_idx:
            issue_idx(j + 3, cur)

    issue_idx(0, B[0])
    wait_idx(B[0])
    issue_gather(B[0])
    issue_idx(1, B[1])
    issue_idx(2, B[2])

    @pl.loop(0, (CPW - 3) // 3)
    def _triple(jj):
        j0 = jj * 3

        @pl.when(jj > 0)
        def _():
            wait_scatter(B[1])
        step(j0, B[0], B[1], False, True, True, True)

        @pl.when(jj > 0)
        def _():
            wait_scatter(B[2])
        step(j0 + 1, B[1], B[2], False, True, True, True)

        step(j0 + 2, B[2], B[0], True, True, True, True)

    step(CPW - 3, B[0], B[1], True, True, True, False)
    step(CPW - 2, B[1], B[2], True, True, True, False)
    step(CPW - 1, B[2], B[0], True, False, False, False)
    wait_scatter(B[1])
    wait_scatter(B[2])

    plsc.subcore_barrier()
    for off, n in _CH:
        r0 = pl.multiple_of(s * RPT + off, 8)
        pltpu.sync_copy(agg_sp.at[pl.ds(r0, n), :], rows0.at[pl.ds(0, n), :])
        pltpu.sync_copy(rows0.at[pl.ds(0, n), :], out_hbm.at[c, pl.ds(r0, n), :])
    d0 = pl.multiple_of(s * RPT, 8)
    pltpu.sync_copy(dw_sp.at[pl.ds(d0, RPT)], dw_hbm.at[c, pl.ds(d0, RPT)])


_edge_call = functools.partial(
    pl.kernel,
    out_type=[jax.ShapeDtypeStruct((NCORES, NPAD, D), _f32),
              jax.ShapeDtypeStruct((NCORES, NPAD), _f32)],
    mesh=plsc.VectorSubcoreMesh(core_axis_name="c", subcore_axis_name="s",
                                num_cores=NCORES, num_subcores=NSUB),
    compiler_params=pltpu.CompilerParams(needs_layout_passes=False),
    scratch_types=[
        pltpu.VMEM_SHARED((NPAD, D), _f32),
        pltpu.VMEM_SHARED((NPAD,), _f32),
        pltpu.VMEM((1, EK), _i32),
        pltpu.VMEM((1, EK), _i32),
        pltpu.VMEM((1, EK), _f32),
        pltpu.VMEM((EK, D), _f32),
        pltpu.VMEM((1, EK), _i32),
        pltpu.VMEM((1, EK), _f32),
        pltpu.VMEM((1, EK), _i32),
        pltpu.VMEM((1, EK), _i32),
        pltpu.VMEM((1, EK), _f32),
        pltpu.VMEM((EK, D), _f32),
        pltpu.VMEM((1, EK), _i32),
        pltpu.VMEM((1, EK), _f32),
        pltpu.VMEM((1, EK), _i32),
        pltpu.VMEM((1, EK), _i32),
        pltpu.VMEM((1, EK), _f32),
        pltpu.VMEM((EK, D), _f32),
        pltpu.VMEM((1, EK), _i32),
        pltpu.VMEM((1, EK), _f32),
        pltpu.SemaphoreType.DMA,
        pltpu.SemaphoreType.DMA,
        pltpu.SemaphoreType.DMA,
    ],
)(_edge_body)


NBLK = NTP // KE


def _final_body(h_hbm, lg_hbm, t_hbm,
                gamma_hbm, ysum_hbm, cnt_hbm,
                m_all, m_fin_sp, ssum_sp, cnt_sp, ysum_sp,
                tpad, va, vb, m_loc, mm1, blk, t5, e5, one5, hbuf):
    w = lax.axis_index("s")
    z16 = jnp.zeros((16,), _f32)
    neg16 = jnp.full((16,), NEG, _f32)
    one16 = jnp.ones((16,), _f32)
    R5 = RPT // KE

    def _for_my_blocks(fn):
        fn(pl.multiple_of(w * KE, KE))

        @pl.when(w < NBLK - NSUB)
        def _():
            fn(pl.multiple_of((w + NSUB) * KE, KE))

    for g in range(KE // 16):
        blk[pl.ds(g * 16, 16)] = z16
    for r in range(KE):
        for d8 in range(D // 16):
            hbuf[r, pl.ds(d8 * 16, 16)] = z16

    def _zero(c0):
        pltpu.sync_copy(blk, ssum_sp.at[pl.ds(c0, KE)])
        pltpu.sync_copy(blk, cnt_sp.at[pl.ds(c0, KE)])
        pltpu.sync_copy(hbuf, ysum_sp.at[pl.ds(c0, KE), :])

    _for_my_blocks(_zero)
    plsc.subcore_barrier()

    for g in range(PADK // 16):
        sl = pl.ds(g * 16, 16)
        tpad[sl] = jnp.full((16,), -1, _i32)
        va[sl] = neg16
        vb[sl] = neg16
    tpad[pl.ds(TOT, 16)] = jnp.full((16,), -3, _i32)
    nb = pl.multiple_of(w * RPT, KE)
    pltpu.sync_copy(t_hbm.at[pl.ds(nb, RPT)], tpad.at[pl.ds(PADK, RPT)])
    pltpu.sync_copy(lg_hbm.at[pl.ds(nb, RPT)], va.at[pl.ds(PADK, RPT)])
    for r5 in range(R5):
        pltpu.sync_copy(t_hbm.at[pl.ds(nb + r5 * KE, KE)], t5.at[r5])

    for g in range(KE // 16):
        one5[0, pl.ds(g * 16, 16)] = one16
    for r5 in range(R5):
        pltpu.sync_copy(one5.at[0], cnt_sp.at[t5.at[r5]], add=True)
        pltpu.sync_copy(h_hbm.at[pl.ds(nb + r5 * KE, KE), :], hbuf)
        pltpu.sync_copy(hbuf, ysum_sp.at[t5.at[r5]], add=True)

    bufs = (va, vb)
    for rnd in range(10):
        k = 1 << rnd
        src = bufs[rnd % 2]
        dst = bufs[(rnd + 1) % 2]

        @pl.loop(0, RPT // 16)
        def _rmax(j, k=k, src=src, dst=dst):
            i0 = PADK + j * 16
            t_c = tpad[pl.ds(i0, 16)]
            t_p = tpad[pl.ds(i0 - k, 16)]
            v_c = src[pl.ds(i0, 16)]
            v_p = src[pl.ds(i0 - k, 16)]
            dst[pl.ds(i0, 16)] = jnp.maximum(
                v_c, jnp.where(t_p == t_c, v_p, NEG))

    for g in range(NTP // 16):
        m_loc[pl.ds(g * 16, 16)] = neg16

    @pl.loop(0, RPT // 16)
    def _scat(j):
        i0 = PADK + j * 16
        t_c = tpad[pl.ds(i0, 16)]
        t_n = tpad[pl.ds(i0 + 1, 16)]
        v = va[pl.ds(i0, 16)]
        plsc.store_scatter(m_loc, [t_c], v, mask=t_c != t_n)

    pltpu.sync_copy(m_loc, m_all.at[pl.ds(pl.multiple_of(w * NTP, KE), NTP)])
    plsc.subcore_barrier()

    def _merge(c0):
        for r in range(NSUB):
            pltpu.sync_copy(m_all.at[pl.ds(r * NTP + c0, KE)],
                            mm1.at[pl.ds(r * KE, KE)])
        for g in range(KE // 16):
            sl = pl.ds(g * 16, 16)
            acc = mm1[sl]
            for r in range(1, NSUB):
                acc = jnp.maximum(acc, mm1[pl.ds(r * KE + g * 16, 16)])
            blk[sl] = acc
        pltpu.sync_copy(blk, m_fin_sp.at[pl.ds(c0, KE)])

    _for_my_blocks(_merge)
    plsc.subcore_barrier()

    pltpu.sync_copy(m_fin_sp, m_loc)
    pltpu.sync_copy(lg_hbm.at[pl.ds(nb, RPT)], vb.at[pl.ds(PADK, RPT)])
    for r5 in range(R5):
        @pl.loop(0, KE // 16)
        def _e(j2, r5=r5):
            i0 = PADK + r5 * KE + j2 * 16
            t_c = tpad[pl.ds(i0, 16)]
            x = vb[pl.ds(i0, 16)]
            mt = plsc.load_gather(m_loc, [t_c])
            e5[r5, pl.ds(j2 * 16, 16)] = jnp.exp(x - mt)

    for r5 in range(R5):
        pltpu.sync_copy(e5.at[r5], ssum_sp.at[t5.at[r5]], add=True)
    plsc.subcore_barrier()

    pltpu.sync_copy(ssum_sp, m_loc)
    for r5 in range(R5):
        @pl.loop(0, KE // 16)
        def _g(j2, r5=r5):
            i0 = PADK + r5 * KE + j2 * 16
            t_c = tpad[pl.ds(i0, 16)]
            sg = plsc.load_gather(m_loc, [t_c])
            sl = pl.ds(j2 * 16, 16)
            e5[r5, sl] = e5[r5, sl] / (sg + 1e-16)
    for r5 in range(R5):
        pltpu.sync_copy(e5.at[r5], gamma_hbm.at[pl.ds(nb + r5 * KE, KE)])

    def _wb(c0):
        pltpu.sync_copy(ysum_sp.at[pl.ds(c0, KE), :], hbuf)
        pltpu.sync_copy(hbuf, ysum_hbm.at[pl.ds(c0, KE), :])
        pltpu.sync_copy(cnt_sp.at[pl.ds(c0, KE)], blk)
        pltpu.sync_copy(blk, cnt_hbm.at[pl.ds(c0, KE)])

    _for_my_blocks(_wb)


_final_call = functools.partial(
    pl.kernel,
    out_type=[jax.ShapeDtypeStruct((NPAD,), _f32),
              jax.ShapeDtypeStruct((NTP, D), _f32),
              jax.ShapeDtypeStruct((NTP,), _f32)],
    mesh=plsc.VectorSubcoreMesh(core_axis_name="c", subcore_axis_name="s",
                                num_cores=1, num_subcores=NSUB),
    compiler_params=pltpu.CompilerParams(needs_layout_passes=False),
    scratch_types=[
        pltpu.VMEM_SHARED((NSUB * NTP,), _f32),
        pltpu.VMEM_SHARED((NTP,), _f32),
        pltpu.VMEM_SHARED((NTP,), _f32),
        pltpu.VMEM_SHARED((NTP,), _f32),
        pltpu.VMEM_SHARED((NTP, D), _f32),
        pltpu.VMEM((TOT + 16,), _i32),
        pltpu.VMEM((TOT,), _f32),
        pltpu.VMEM((TOT,), _f32),
        pltpu.VMEM((NTP,), _f32),
        pltpu.VMEM((NSUB * KE,), _f32),
        pltpu.VMEM((KE,), _f32),
        pltpu.VMEM((RPT // KE, KE), _i32),
        pltpu.VMEM((RPT // KE, KE), _f32),
        pltpu.VMEM((1, KE), _f32),
        pltpu.VMEM((KE, D), _f32),
    ],
)(_final_body)


def kernel(y, edge_index, edge_weight, transmitters_index,
           l1_W1, l1_b1, l1_W2, l1_W3, l1_b3,
           l2_W1, l2_b1, l2_W2, l2_W3, l2_b3,
           Wg, Wp):
    src = edge_index[0]
    dst = edge_index[1]
    pad_e = EPAD - E
    srcp = jnp.concatenate([src, jnp.zeros((pad_e,), _i32)])
    dstp = jnp.concatenate([dst, jnp.zeros((pad_e,), _i32)])
    ewp = jnp.concatenate([edge_weight, jnp.zeros((pad_e,), _f32)])
    tpad = jnp.concatenate(
        [transmitters_index, jnp.full((NPAD - N,), TRASH, _i32)])

    a1, bv1, c1 = _tc_mm3(y, l1_W1, l1_W2, l1_W3, l1_b1, l1_b3)
    agg1, dw = _edge_call(a1, srcp, dstp, ewp)
    dwr = dw.reshape(NCORES, NPAD, 1)

    a2, bv2, c2 = _tc_hmm3(agg1, dwr, bv1, c1,
                           l2_W1, l2_W2, l2_W3, l2_b1, l2_b3)
    agg2, _ = _edge_call(a2, srcp, dstp, ewp)

    h2, logits = _tc_hlog(agg2, dwr, bv2, c2, Wg)
    h2p = jnp.pad(h2, ((0, NPAD - N), (0, 0)))
    lgp = jnp.pad(logits.reshape(-1), (0, NPAD - N))

    gamma_flat, ysum, cnt = _final_call(h2p, lgp, tpad)
    gamma = gamma_flat[:N].reshape(N, 1)
    p = _tc_p(ysum, cnt.reshape(NTP, 1), Wp)[:NT]
    return (p, gamma)


# speedup vs baseline: 2.8599x; 1.3180x over previous
import functools

import jax
import jax.numpy as jnp
from jax import lax
from jax.experimental import pallas as pl
from jax.experimental.pallas import tpu as pltpu
from jax.experimental.pallas import tpu_sc as plsc

N = 10000
E = 320000
D = 128
NT = 2500
P_MAX = 10.0
TAU = 1.0
NEG = -1e30

NCORES = 2
NSUB = 16
NWORK = NCORES * NSUB
KE = 128
EK = 112
CPW0 = 63
CPW1 = 117
EPW0 = EK * CPW0
EPW1 = EK * CPW1
C1B = NSUB * EPW0
EPAD = NSUB * (EPW0 + EPW1)
NPAD = 10240
RPT = 640

NTP = 2560
TRASH = 2559
CT = 160
PADK = 1024
TOT = PADK + RPT

_f32 = jnp.float32
_i32 = jnp.int32


_RB = 2000


def _dotT(x, w):
    return lax.dot_general(x.astype(jnp.bfloat16), w.astype(jnp.bfloat16),
                           (((1,), (1,)), ((), ())),
                           preferred_element_type=_f32)


def _dotvT(x, w):
    xb = x.astype(jnp.bfloat16).astype(_f32)
    wb = w.astype(jnp.bfloat16).astype(_f32)
    return jnp.sum(xb * wb, axis=1, keepdims=True)


def _leaky(h):
    return jnp.where(h >= 0, h, 0.01 * h)


def _mm3_body(x_ref, w1_ref, w2_ref, w3_ref, b1_ref, b3_ref,
              a_ref, b_ref, c_ref):
    x = x_ref[...]
    a_ref[...] = _dotT(x, w1_ref[...]) + b1_ref[...]
    b_ref[...] = _dotT(x, w2_ref[...])
    c_ref[...] = _dotT(x, w3_ref[...]) + b3_ref[...]


def _tc_mm3(y, w1, w2, w3, b1, b3):
    spec_x = pl.BlockSpec((_RB, D), lambda i: (i, 0))
    spec_w = pl.BlockSpec((D, D), lambda i: (0, 0))
    spec_b = pl.BlockSpec((1, D), lambda i: (0, 0))
    out = jax.ShapeDtypeStruct((N, D), _f32)
    return pl.pallas_call(
        _mm3_body,
        grid=(N // _RB,),
        in_specs=[spec_x, spec_w, spec_w, spec_w, spec_b, spec_b],
        out_specs=[spec_x, spec_x, spec_x],
        out_shape=[out, out, out],
    )(y, w1, w2, w3, b1.reshape(1, D), b3.reshape(1, D))


def _hmm3_body(s0_ref, s1_ref, dw0_ref, dw1_ref, bv_ref, c_ref,
               w1_ref, w2_ref, w3_ref, b1_ref, b3_ref,
               a_ref, b_ref, c2_ref):
    dw = dw0_ref[0] + dw1_ref[0]
    h = s0_ref[0] + s1_ref[0] - dw * bv_ref[...] + c_ref[...]
    h = _leaky(h)
    a_ref[...] = _dotT(h, w1_ref[...]) + b1_ref[...]
    b_ref[...] = _dotT(h, w2_ref[...])
    c2_ref[...] = _dotT(h, w3_ref[...]) + b3_ref[...]


def _tc_hmm3(aggs, dws, bv, c, w1, w2, w3, b1, b3):
    spec_x = pl.BlockSpec((_RB, D), lambda i: (i, 0))
    spec_s0 = pl.BlockSpec((1, _RB, D), lambda i: (0, i, 0))
    spec_s1 = pl.BlockSpec((1, _RB, D), lambda i: (1, i, 0))
    spec_d0 = pl.BlockSpec((1, _RB, 1), lambda i: (0, i, 0))
    spec_d1 = pl.BlockSpec((1, _RB, 1), lambda i: (1, i, 0))
    spec_w = pl.BlockSpec((D, D), lambda i: (0, 0))
    spec_b = pl.BlockSpec((1, D), lambda i: (0, 0))
    out = jax.ShapeDtypeStruct((N, D), _f32)
    return pl.pallas_call(
        _hmm3_body,
        grid=(N // _RB,),
        in_specs=[spec_s0, spec_s1, spec_d0, spec_d1, spec_x, spec_x,
                  spec_w, spec_w, spec_w, spec_b, spec_b],
        out_specs=[spec_x, spec_x, spec_x],
        out_shape=[out, out, out],
    )(aggs, aggs, dws, dws, bv, c,
      w1, w2, w3, b1.reshape(1, D), b3.reshape(1, D))


def _hlog_body(s0_ref, s1_ref, dw0_ref, dw1_ref, bv_ref, c_ref, wg_ref,
               h_ref, lg_ref):
    dw = dw0_ref[0] + dw1_ref[0]
    h = s0_ref[0] + s1_ref[0] - dw * bv_ref[...] + c_ref[...]
    h = _leaky(h)
    h_ref[...] = h
    lg_ref[...] = _dotvT(h, wg_ref[...]) * (1.0 / TAU)


def _tc_hlog(aggs, dws, bv, c, wg):
    spec_x = pl.BlockSpec((_RB, D), lambda i: (i, 0))
    spec_s0 = pl.BlockSpec((1, _RB, D), lambda i: (0, i, 0))
    spec_s1 = pl.BlockSpec((1, _RB, D), lambda i: (1, i, 0))
    spec_d0 = pl.BlockSpec((1, _RB, 1), lambda i: (0, i, 0))
    spec_d1 = pl.BlockSpec((1, _RB, 1), lambda i: (1, i, 0))
    spec_wg = pl.BlockSpec((1, D), lambda i: (0, 0))
    spec_lg = pl.BlockSpec((_RB, 1), lambda i: (i, 0))
    return pl.pallas_call(
        _hlog_body,
        grid=(N // _RB,),
        in_specs=[spec_s0, spec_s1, spec_d0, spec_d1, spec_x, spec_x, spec_wg],
        out_specs=[spec_x, spec_lg],
        out_shape=[jax.ShapeDtypeStruct((N, D), _f32),
                   jax.ShapeDtypeStruct((N, 1), _f32)],
    )(aggs, aggs, dws, dws, bv, c, wg)


def _p_body(ys_ref, cnt_ref, wp_ref, p_ref):
    tx = ys_ref[...] / jnp.maximum(cnt_ref[...], 1.0)
    z = _dotvT(tx, wp_ref[...])
    p_ref[...] = P_MAX * jax.nn.sigmoid(z)


def _tc_p(ysum, cnt, wp):
    return pl.pallas_call(
        _p_body,
        grid=(1,),
        in_specs=[pl.BlockSpec((NTP, D), lambda i: (0, 0)),
                  pl.BlockSpec((NTP, 1), lambda i: (0, 0)),
                  pl.BlockSpec((1, D), lambda i: (0, 0))],
        out_specs=pl.BlockSpec((NTP, 1), lambda i: (0, 0)),
        out_shape=jax.ShapeDtypeStruct((NTP, 1), _f32),
    )(ysum, cnt, wp)


def _edge_body(a_hbm, src_hbm, dst_hbm, ew_hbm,
               out_hbm, dw_hbm,
               agg_sp, dw_sp,
               sidx0, didx0, ewv0, rows0, pdix0, pew0,
               sidx1, didx1, ewv1, rows1, pdix1, pew1,
               sidx2, didx2, ewv2, rows2, pdix2, pew2,
               gsem, ssem, isem):
    c = lax.axis_index("c")
    s = lax.axis_index("s")
    z16 = jnp.zeros((16,), _f32)
    ebase = jnp.where(c == 0, s * EPW0, C1B + s * EPW1)
    cpw = jnp.where(c == 0, CPW0, CPW1)
    B = ((sidx0, didx0, ewv0, rows0, pdix0, pew0),
         (sidx1, didx1, ewv1, rows1, pdix1, pew1),
         (sidx2, didx2, ewv2, rows2, pdix2, pew2))
    _CH = [(i * EK, EK) for i in range(RPT // EK)]
    if RPT % EK:
        _CH.append((RPT // EK * EK, RPT % EK))

    for r in range(EK):
        for d8 in range(D // 16):
            rows0[r, pl.ds(d8 * 16, 16)] = z16
    for d8 in range(EK // 16):
        ewv0[0, pl.ds(d8 * 16, 16)] = z16
    for off, n in _CH:
        r0 = pl.multiple_of(s * RPT + off, 8)
        pltpu.sync_copy(rows0.at[pl.ds(0, n), :], agg_sp.at[pl.ds(r0, n), :])
        pltpu.sync_copy(ewv0.at[0, pl.ds(0, n)], dw_sp.at[pl.ds(r0, n)])
    plsc.subcore_barrier()

    def issue_idx(j, b):
        base = pl.multiple_of(ebase + j * EK, 8)
        pltpu.async_copy(src_hbm.at[pl.ds(base, EK)], b[0].at[0], isem)
        pltpu.async_copy(dst_hbm.at[pl.ds(base, EK)], b[1].at[0], isem)
        pltpu.async_copy(ew_hbm.at[pl.ds(base, EK)], b[2].at[0], isem)

    def wait_idx(b):
        pltpu.make_async_copy(src_hbm.at[pl.ds(0, EK)], b[0].at[0], isem).wait()
        pltpu.make_async_copy(dst_hbm.at[pl.ds(0, EK)], b[1].at[0], isem).wait()
        pltpu.make_async_copy(ew_hbm.at[pl.ds(0, EK)], b[2].at[0], isem).wait()

    def issue_gather(b):
        pltpu.async_copy(a_hbm.at[b[0].at[0]], b[3], gsem)

    def wait_gather(b):
        pltpu.make_async_copy(a_hbm.at[b[0].at[0]], b[3], gsem).wait()

    def scale(b):
        ewv, rows = b[2], b[3]

        @pl.loop(0, EK // 16)
        def _sc(g):
            ew16 = ewv[0, pl.ds(g * 16, 16)]
            for lane in range(16):
                e = g * 16 + lane
                sc = ew16[lane]
                for d8 in range(D // 16):
                    sl = pl.ds(d8 * 16, 16)
                    rows[e, sl] = rows[e, sl] * sc

    def copy_priv(b):
        for g in range(EK // 16):
            sl = pl.ds(g * 16, 16)
            b[4][0, sl] = b[1][0, sl]
            b[5][0, sl] = b[2][0, sl]

    def issue_scatter(b):
        pltpu.async_copy(b[3], agg_sp.at[b[4].at[0]], ssem, add=True)
        pltpu.async_copy(b[5].at[0], dw_sp.at[b[4].at[0]], ssem, add=True)

    def wait_scatter(b):
        pltpu.make_async_copy(b[3], agg_sp.at[b[4].at[0]], ssem).wait()
        pltpu.make_async_copy(b[5].at[0], dw_sp.at[b[4].at[0]], ssem).wait()

    def step(j, cur, nxt, w_scat, w_idx, i_gath, i_idx):
        if w_scat:
            wait_scatter(nxt)
        if w_idx:
            wait_idx(nxt)
        if i_gath:
            issue_gather(nxt)
        wait_gather(cur)
        scale(cur)
        copy_priv(cur)
        issue_scatter(cur)
        if i_idx:
            issue_idx(j + 3, cur)

    issue_idx(0, B[0])
    wait_idx(B[0])
    issue_gather(B[0])
    issue_idx(1, B[1])
    issue_idx(2, B[2])

    @pl.loop(0, (cpw - 3) // 3)
    def _triple(jj):
        j0 = jj * 3

        @pl.when(jj > 0)
        def _():
            wait_scatter(B[1])
        step(j0, B[0], B[1], False, True, True, True)

        @pl.when(jj > 0)
        def _():
            wait_scatter(B[2])
        step(j0 + 1, B[1], B[2], False, True, True, True)

        step(j0 + 2, B[2], B[0], True, True, True, True)

    step(cpw - 3, B[0], B[1], True, True, True, False)
    step(cpw - 2, B[1], B[2], True, True, True, False)
    step(cpw - 1, B[2], B[0], True, False, False, False)
    wait_scatter(B[1])
    wait_scatter(B[2])

    plsc.subcore_barrier()
    for off, n in _CH:
        r0 = pl.multiple_of(s * RPT + off, 8)
        pltpu.sync_copy(agg_sp.at[pl.ds(r0, n), :], rows0.at[pl.ds(0, n), :])
        pltpu.sync_copy(rows0.at[pl.ds(0, n), :], out_hbm.at[c, pl.ds(r0, n), :])
    d0 = pl.multiple_of(s * RPT, 8)
    pltpu.sync_copy(dw_sp.at[pl.ds(d0, RPT)], dw_hbm.at[c, pl.ds(d0, RPT)])


_edge_call = functools.partial(
    pl.kernel,
    out_type=[jax.ShapeDtypeStruct((NCORES, NPAD, D), _f32),
              jax.ShapeDtypeStruct((NCORES, NPAD), _f32)],
    mesh=plsc.VectorSubcoreMesh(core_axis_name="c", subcore_axis_name="s",
                                num_cores=NCORES, num_subcores=NSUB),
    compiler_params=pltpu.CompilerParams(needs_layout_passes=False),
    scratch_types=[
        pltpu.VMEM_SHARED((NPAD, D), _f32),
        pltpu.VMEM_SHARED((NPAD,), _f32),
        pltpu.VMEM((1, EK), _i32),
        pltpu.VMEM((1, EK), _i32),
        pltpu.VMEM((1, EK), _f32),
        pltpu.VMEM((EK, D), _f32),
        pltpu.VMEM((1, EK), _i32),
        pltpu.VMEM((1, EK), _f32),
        pltpu.VMEM((1, EK), _i32),
        pltpu.VMEM((1, EK), _i32),
        pltpu.VMEM((1, EK), _f32),
        pltpu.VMEM((EK, D), _f32),
        pltpu.VMEM((1, EK), _i32),
        pltpu.VMEM((1, EK), _f32),
        pltpu.VMEM((1, EK), _i32),
        pltpu.VMEM((1, EK), _i32),
        pltpu.VMEM((1, EK), _f32),
        pltpu.VMEM((EK, D), _f32),
        pltpu.VMEM((1, EK), _i32),
        pltpu.VMEM((1, EK), _f32),
        pltpu.SemaphoreType.DMA,
        pltpu.SemaphoreType.DMA,
        pltpu.SemaphoreType.DMA,
    ],
)(_edge_body)


NBLK = NTP // KE


def _final_body(h_hbm, lg_hbm, t_hbm,
                gamma_hbm, ysum_hbm, cnt_hbm,
                m_all, m_fin_sp, ssum_sp, cnt_sp, ysum_sp,
                tpad, va, vb, m_loc, mm1, blk, t5, e5, one5, hbuf):
    w = lax.axis_index("s")
    z16 = jnp.zeros((16,), _f32)
    neg16 = jnp.full((16,), NEG, _f32)
    one16 = jnp.ones((16,), _f32)
    R5 = RPT // KE

    def _for_my_blocks(fn):
        fn(pl.multiple_of(w * KE, KE))

        @pl.when(w < NBLK - NSUB)
        def _():
            fn(pl.multiple_of((w + NSUB) * KE, KE))

    for g in range(KE // 16):
        blk[pl.ds(g * 16, 16)] = z16
    for r in range(KE):
        for d8 in range(D // 16):
            hbuf[r, pl.ds(d8 * 16, 16)] = z16

    def _zero(c0):
        pltpu.sync_copy(blk, ssum_sp.at[pl.ds(c0, KE)])
        pltpu.sync_copy(blk, cnt_sp.at[pl.ds(c0, KE)])
        pltpu.sync_copy(hbuf, ysum_sp.at[pl.ds(c0, KE), :])

    _for_my_blocks(_zero)
    plsc.subcore_barrier()

    for g in range(PADK // 16):
        sl = pl.ds(g * 16, 16)
        tpad[sl] = jnp.full((16,), -1, _i32)
        va[sl] = neg16
        vb[sl] = neg16
    tpad[pl.ds(TOT, 16)] = jnp.full((16,), -3, _i32)
    nb = pl.multiple_of(w * RPT, KE)
    pltpu.sync_copy(t_hbm.at[pl.ds(nb, RPT)], tpad.at[pl.ds(PADK, RPT)])
    pltpu.sync_copy(lg_hbm.at[pl.ds(nb, RPT)], va.at[pl.ds(PADK, RPT)])
    for r5 in range(R5):
        pltpu.sync_copy(t_hbm.at[pl.ds(nb + r5 * KE, KE)], t5.at[r5])

    for g in range(KE // 16):
        one5[0, pl.ds(g * 16, 16)] = one16
    for r5 in range(R5):
        pltpu.sync_copy(one5.at[0], cnt_sp.at[t5.at[r5]], add=True)
        pltpu.sync_copy(h_hbm.at[pl.ds(nb + r5 * KE, KE), :], hbuf)
        pltpu.sync_copy(hbuf, ysum_sp.at[t5.at[r5]], add=True)

    bufs = (va, vb)
    for rnd in range(10):
        k = 1 << rnd
        src = bufs[rnd % 2]
        dst = bufs[(rnd + 1) % 2]

        @pl.loop(0, RPT // 16)
        def _rmax(j, k=k, src=src, dst=dst):
            i0 = PADK + j * 16
            t_c = tpad[pl.ds(i0, 16)]
            t_p = tpad[pl.ds(i0 - k, 16)]
            v_c = src[pl.ds(i0, 16)]
            v_p = src[pl.ds(i0 - k, 16)]
            dst[pl.ds(i0, 16)] = jnp.maximum(
                v_c, jnp.where(t_p == t_c, v_p, NEG))

    for g in range(NTP // 16):
        m_loc[pl.ds(g * 16, 16)] = neg16

    @pl.loop(0, RPT // 16)
    def _scat(j):
        i0 = PADK + j * 16
        t_c = tpad[pl.ds(i0, 16)]
        t_n = tpad[pl.ds(i0 + 1, 16)]
        v = va[pl.ds(i0, 16)]
        plsc.store_scatter(m_loc, [t_c], v, mask=t_c != t_n)

    pltpu.sync_copy(m_loc, m_all.at[pl.ds(pl.multiple_of(w * NTP, KE), NTP)])
    plsc.subcore_barrier()

    def _merge(c0):
        for r in range(NSUB):
            pltpu.sync_copy(m_all.at[pl.ds(r * NTP + c0, KE)],
                            mm1.at[pl.ds(r * KE, KE)])
        for g in range(KE // 16):
            sl = pl.ds(g * 16, 16)
            acc = mm1[sl]
            for r in range(1, NSUB):
                acc = jnp.maximum(acc, mm1[pl.ds(r * KE + g * 16, 16)])
            blk[sl] = acc
        pltpu.sync_copy(blk, m_fin_sp.at[pl.ds(c0, KE)])

    _for_my_blocks(_merge)
    plsc.subcore_barrier()

    pltpu.sync_copy(m_fin_sp, m_loc)
    pltpu.sync_copy(lg_hbm.at[pl.ds(nb, RPT)], vb.at[pl.ds(PADK, RPT)])
    for r5 in range(R5):
        @pl.loop(0, KE // 16)
        def _e(j2, r5=r5):
            i0 = PADK + r5 * KE + j2 * 16
            t_c = tpad[pl.ds(i0, 16)]
            x = vb[pl.ds(i0, 16)]
            mt = plsc.load_gather(m_loc, [t_c])
            e5[r5, pl.ds(j2 * 16, 16)] = jnp.exp(x - mt)

    for r5 in range(R5):
        pltpu.sync_copy(e5.at[r5], ssum_sp.at[t5.at[r5]], add=True)
    plsc.subcore_barrier()

    pltpu.sync_copy(ssum_sp, m_loc)
    for r5 in range(R5):
        @pl.loop(0, KE // 16)
        def _g(j2, r5=r5):
            i0 = PADK + r5 * KE + j2 * 16
            t_c = tpad[pl.ds(i0, 16)]
            sg = plsc.load_gather(m_loc, [t_c])
            sl = pl.ds(j2 * 16, 16)
            e5[r5, sl] = e5[r5, sl] / (sg + 1e-16)
    for r5 in range(R5):
        pltpu.sync_copy(e5.at[r5], gamma_hbm.at[pl.ds(nb + r5 * KE, KE)])

    def _wb(c0):
        pltpu.sync_copy(ysum_sp.at[pl.ds(c0, KE), :], hbuf)
        pltpu.sync_copy(hbuf, ysum_hbm.at[pl.ds(c0, KE), :])
        pltpu.sync_copy(cnt_sp.at[pl.ds(c0, KE)], blk)
        pltpu.sync_copy(blk, cnt_hbm.at[pl.ds(c0, KE)])

    _for_my_blocks(_wb)


_final_call = functools.partial(
    pl.kernel,
    out_type=[jax.ShapeDtypeStruct((NPAD,), _f32),
              jax.ShapeDtypeStruct((NTP, D), _f32),
              jax.ShapeDtypeStruct((NTP,), _f32)],
    mesh=plsc.VectorSubcoreMesh(core_axis_name="c", subcore_axis_name="s",
                                num_cores=1, num_subcores=NSUB),
    compiler_params=pltpu.CompilerParams(needs_layout_passes=False),
    scratch_types=[
        pltpu.VMEM_SHARED((NSUB * NTP,), _f32),
        pltpu.VMEM_SHARED((NTP,), _f32),
        pltpu.VMEM_SHARED((NTP,), _f32),
        pltpu.VMEM_SHARED((NTP,), _f32),
        pltpu.VMEM_SHARED((NTP, D), _f32),
        pltpu.VMEM((TOT + 16,), _i32),
        pltpu.VMEM((TOT,), _f32),
        pltpu.VMEM((TOT,), _f32),
        pltpu.VMEM((NTP,), _f32),
        pltpu.VMEM((NSUB * KE,), _f32),
        pltpu.VMEM((KE,), _f32),
        pltpu.VMEM((RPT // KE, KE), _i32),
        pltpu.VMEM((RPT // KE, KE), _f32),
        pltpu.VMEM((1, KE), _f32),
        pltpu.VMEM((KE, D), _f32),
    ],
)(_final_body)


def kernel(y, edge_index, edge_weight, transmitters_index,
           l1_W1, l1_b1, l1_W2, l1_W3, l1_b3,
           l2_W1, l2_b1, l2_W2, l2_W3, l2_b3,
           Wg, Wp):
    src = edge_index[0]
    dst = edge_index[1]
    pad_e = EPAD - E
    spread = (jnp.arange(pad_e, dtype=_i32) * 97) % N
    srcp = jnp.concatenate([src, spread])
    dstp = jnp.concatenate([dst, spread])
    ewp = jnp.concatenate([edge_weight, jnp.zeros((pad_e,), _f32)])
    tpad = jnp.concatenate(
        [transmitters_index, jnp.full((NPAD - N,), TRASH, _i32)])

    a1, bv1, c1 = _tc_mm3(y, l1_W1, l1_W2, l1_W3, l1_b1, l1_b3)
    agg1, dw = _edge_call(a1, srcp, dstp, ewp)
    dwr = dw.reshape(NCORES, NPAD, 1)

    a2, bv2, c2 = _tc_hmm3(agg1, dwr, bv1, c1,
                           l2_W1, l2_W2, l2_W3, l2_b1, l2_b3)
    agg2, _ = _edge_call(a2, srcp, dstp, ewp)

    h2, logits = _tc_hlog(agg2, dwr, bv2, c2, Wg)
    h2p = jnp.pad(h2, ((0, NPAD - N), (0, 0)))
    lgp = jnp.pad(logits.reshape(-1), (0, NPAD - N))

    gamma_flat, ysum, cnt = _final_call(h2p, lgp, tpad)
    gamma = gamma_flat[:N].reshape(N, 1)
    p = _tc_p(ysum, cnt.reshape(NTP, 1), Wp)[:NT]
    return (p, gamma)
